# baseline (speedup 1.0000x reference)
import dataclasses
import functools

import jax
import jax.numpy as jnp
from jax import lax
from jax.experimental import pallas as pl
from jax.experimental.pallas import tpu as pltpu
from jax.experimental.pallas import tpu_sc as plsc

N = 10000
D = 128
DP = D // 2
CHUNK = 80
N_TILES = 32
SC_TILES = 16
ROWS_PER_TILE = 632
N_PAD = SC_TILES * ROWS_PER_TILE
ROW_BLOCK = 1000


NBUF = 3
MSLOTS = 6
LEAD = NBUF - 1
CPT0 = 162
CPT1 = 90


def _expand_scale(pin, rout, wrow_ref):
    @pl.loop(0, CHUNK, unroll=2)
    def _edge(e):
        wbits = plsc.load_gather(wrow_ref, [jnp.full((16,), e, jnp.int32)])
        w = plsc.bitcast(wbits, jnp.float32)
        for g in range(DP // 16):
            x = pin[e, pl.ds(g * 16, 16)]
            flo = plsc.bitcast(lax.shift_left(x, 16), jnp.float32) * w
            fhi = plsc.bitcast(jnp.bitwise_and(x, jnp.int32(-65536)),
                               jnp.float32) * w
            rout[e, pl.ds(g * 16, 16)] = flo
            rout[e, pl.ds(DP + g * 16, 16)] = fhi


def _sc_body(hlin_hbm, meta_hbm, zeros_hbm, out_hbm,
             p0, p1, p2, r0, r1, r2, m0, m1, m2, m3, m4, m5, acc,
             g0, g1, g2, s0, s1, s2, n0, n1, n2, n3, n4, n5,
             ):
    cid = lax.axis_index("c")
    sid = lax.axis_index("s")
    pin = (p0, p1, p2)
    rows = (r0, r1, r2)
    meta = (m0, m1, m2, m3, m4, m5)
    gsem = (g0, g1, g2)
    ssem = (s0, s1, s2)
    msem = (n0, n1, n2, n3, n4, n5)
    cpt = jnp.where(cid == 0, CPT0, CPT1)
    base = jnp.where(cid == 0, sid * CPT0, SC_TILES * CPT0 + sid * CPT1)

    row0 = sid * ROWS_PER_TILE
    pltpu.sync_copy(zeros_hbm, acc.at[pl.ds(row0, ROWS_PER_TILE)])
    plsc.subcore_barrier()

    def start_meta(ms, c):
        pltpu.async_copy(meta_hbm.at[base + c], meta[ms], msem[ms])

    def wait_meta(ms, c):
        pltpu.make_async_copy(meta_hbm.at[base + c], meta[ms], msem[ms]).wait()

    def start_gather(b, ms, c):
        pltpu.async_copy(hlin_hbm.at[meta[ms].at[0]], pin[b], gsem[b])

    def wait_gather(b, ms):
        pltpu.make_async_copy(hlin_hbm.at[meta[ms].at[0]], pin[b],
                              gsem[b]).wait()

    def start_scatter(b, ms, c):
        pltpu.async_copy(rows[b], acc.at[meta[ms].at[1]], ssem[b], add=True)

    def wait_scatter(b, ms):
        pltpu.make_async_copy(rows[b], acc.at[meta[ms].at[1]], ssem[b]).wait()

    for ms in range(MSLOTS):
        pltpu.sync_copy(meta_hbm.at[base + ms], meta[ms])
    for b in range(LEAD):
        start_gather(b, b, b)

    @pl.loop(0, cpt, step=MSLOTS)
    def _group(k):
        for i in range(MSLOTS):
            c = k + i
            b = i % NBUF
            ms = i
            wait_gather(b, ms)
            _expand_scale(pin[b], rows[b], meta[ms].at[2])
            start_scatter(b, ms, c)

            bn = (b + NBUF - 1) % NBUF
            msn = (i + LEAD) % MSLOTS
            msp = (i + MSLOTS - 1) % MSLOTS

            @pl.when(c == 0)
            def _():
                start_gather(bn, msn, LEAD)

            @pl.when(jnp.logical_and(c >= 1, c + LEAD < cpt))
            def _():
                wait_scatter(bn, msp)

                @pl.when(c + MSLOTS - 1 < cpt)
                def _():
                    start_meta(msp, c + MSLOTS - 1)

                @pl.when(c >= MSLOTS - LEAD)
                def _():
                    wait_meta(msn, c + LEAD)

                start_gather(bn, msn, c + LEAD)

    for b in range(NBUF):
        wait_scatter(b, b)

    plsc.subcore_barrier()
    pltpu.sync_copy(acc.at[pl.ds(row0, ROWS_PER_TILE)],
                    out_hbm.at[cid, pl.ds(row0, ROWS_PER_TILE)])


_SC_PARAMS = pltpu.CompilerParams()
for _f, _v in (("needs_layout_passes", False), ("use_tc_tiling_on_sc", False)):
    if _f in pltpu.CompilerParams.__dataclass_fields__:
        _SC_PARAMS = dataclasses.replace(_SC_PARAMS, **{_f: _v})


def _sc_segment(hlin_packed, meta, zeros_rows):
    kern = pl.kernel(
        _sc_body,
        out_type=jax.ShapeDtypeStruct((2, N_PAD, D), jnp.float32),
        mesh=plsc.VectorSubcoreMesh(core_axis_name="c", subcore_axis_name="s"),
        scratch_types=(
            [pltpu.VMEM((CHUNK, DP), jnp.int32) for _ in range(NBUF)]
            + [pltpu.VMEM((CHUNK, D), jnp.float32) for _ in range(NBUF)]
            + [pltpu.VMEM((3, CHUNK), jnp.int32) for _ in range(MSLOTS)]
            + [pltpu.VMEM_SHARED((N_PAD, D), jnp.float32)]
            + [pltpu.SemaphoreType.DMA for _ in range(NBUF + NBUF + MSLOTS)]
        ),
        compiler_params=_SC_PARAMS,
    )
    return kern(hlin_packed, meta, zeros_rows)


def _mm(a, w):
    return lax.dot_general(a, w, (((1,), (1,)), ((), ())),
                           preferred_element_type=jnp.float32)


def _pack_rows(h):
    u = lax.bitcast_convert_type(h, jnp.int32) + jnp.int32(0x8000)
    lo = lax.shift_right_logical(u[:, :DP], 16)
    hi = jnp.bitwise_and(u[:, DP:], jnp.int32(-65536))
    return jnp.bitwise_or(lo, hi)


def _in_body(x_ref, wi_ref, bi_ref, w0_ref, b0_ref, t_ref, hlin_ref, hid_ref):
    h = _mm(x_ref[...], wi_ref[...]) + bi_ref[...]
    hid_ref[...] = h * t_ref[0]
    hlin_ref[...] = _pack_rows(_mm(h, w0_ref[...]) + b0_ref[...])


def _mid_body(p_ref, w_ref, b_ref, hin_ref, t_ref, hlin_ref, hout_ref, *, ti):
    h = jnp.maximum(p_ref[0] + p_ref[1], 0.0)
    hout_ref[...] = hin_ref[...] + h * t_ref[ti]
    hlin_ref[...] = _pack_rows(_mm(h, w_ref[...]) + b_ref[...])


def _fin_body(p_ref, w_ref, b_ref, hin_ref, t_ref, out_ref, *, ti):
    h = jnp.maximum(p_ref[0] + p_ref[1], 0.0)
    hidden = hin_ref[...] + h * t_ref[ti]
    out_ref[...] = _mm(hidden, w_ref[...]) + b_ref[...]


_W_SPEC = pl.BlockSpec((D, D), lambda i: (0, 0))
_B_SPEC = pl.BlockSpec((1, D), lambda i: (0, 0))
_ROW_SPEC = pl.BlockSpec((ROW_BLOCK, D), lambda i: (i, 0))
_PKD_SPEC = pl.BlockSpec((ROW_BLOCK, DP), lambda i: (i, 0))
_P_SPEC = pl.BlockSpec((2, ROW_BLOCK, D), lambda i: (0, i, 0))
_T_SPEC = pl.BlockSpec(memory_space=pltpu.SMEM)


def _in_call(x, W_in, b_in, W0, b0, temp):
    n = x.shape[0]
    return pl.pallas_call(
        _in_body,
        grid=(n // ROW_BLOCK,),
        in_specs=[_ROW_SPEC, _W_SPEC, _B_SPEC, _W_SPEC, _B_SPEC, _T_SPEC],
        out_specs=[_PKD_SPEC, _ROW_SPEC],
        out_shape=[jax.ShapeDtypeStruct((n, DP), jnp.int32),
                   jax.ShapeDtypeStruct((n, D), jnp.float32)],
    )(x, W_in, b_in, W0, b0, temp)


def _mid_call(parts, W, b, hidden, temp, ti):
    n = hidden.shape[0]
    return pl.pallas_call(
        functools.partial(_mid_body, ti=ti),
        grid=(n // ROW_BLOCK,),
        in_specs=[_P_SPEC, _W_SPEC, _B_SPEC, _ROW_SPEC, _T_SPEC],
        out_specs=[_PKD_SPEC, _ROW_SPEC],
        out_shape=[jax.ShapeDtypeStruct((n, DP), jnp.int32),
                   jax.ShapeDtypeStruct((n, D), jnp.float32)],
    )(parts, W, b, hidden, temp)


def _fin_call(parts, W, b, hidden, temp, ti):
    n = hidden.shape[0]
    return pl.pallas_call(
        functools.partial(_fin_body, ti=ti),
        grid=(n // ROW_BLOCK,),
        in_specs=[_P_SPEC, _W_SPEC, _B_SPEC, _ROW_SPEC, _T_SPEC],
        out_specs=_ROW_SPEC,
        out_shape=jax.ShapeDtypeStruct((n, D), jnp.float32),
    )(parts, W, b, hidden, temp)


def kernel(x, edge_index, edge_w, W_in, b_in, W_layers, b_layers, W_out, b_out, temp):
    K = W_layers.shape[0]
    E = edge_index.shape[1]
    total_chunks = SC_TILES * (CPT0 + CPT1)
    e_pad = total_chunks * CHUNK
    pad = e_pad - E
    src = jnp.concatenate([edge_index[0], jnp.zeros((pad,), jnp.int32)])
    dst = jnp.concatenate([edge_index[1], jnp.zeros((pad,), jnp.int32)])
    wp = jnp.concatenate([edge_w, jnp.zeros((pad,), jnp.float32)])
    meta = jnp.stack([src, dst, lax.bitcast_convert_type(wp, jnp.int32)])
    meta = meta.reshape(3, total_chunks, CHUNK).transpose(1, 0, 2)
    zeros_rows = jnp.zeros((ROWS_PER_TILE, D), jnp.float32)

    hlin, hidden = _in_call(x, W_in, b_in.reshape(1, D),
                            W_layers[0], b_layers[0].reshape(1, D), temp)
    out = None
    for i in range(K):
        parts = _sc_segment(hlin, meta, zeros_rows)
        if i < K - 1:
            hlin, hidden = _mid_call(parts, W_layers[i + 1],
                                     b_layers[i + 1].reshape(1, D),
                                     hidden, temp, i + 1)
        else:
            out = _fin_call(parts, W_out, b_out.reshape(1, D),
                            hidden, temp, i + 1)
    return out


# speedup vs baseline: 1.5793x; 1.5793x over previous
import dataclasses
import functools

import jax
import jax.numpy as jnp
from jax import lax
from jax.experimental import pallas as pl
from jax.experimental.pallas import tpu as pltpu
from jax.experimental.pallas import tpu_sc as plsc

N = 10000
D = 128
DP = D // 2
CHUNK = 80
N_TILES = 32
SC_TILES = 16
ROWS_PER_TILE = 632
N_PAD = SC_TILES * ROWS_PER_TILE
ROW_BLOCK = 1000


NBUF = 3
MSLOTS = 6
LEAD = NBUF - 1
CPT0 = 168
CPT1 = 84


def _expand_scale(pin, rout, wrow_ref):
    @pl.loop(0, CHUNK, unroll=2)
    def _edge(e):
        wbits = plsc.load_gather(wrow_ref, [jnp.full((16,), e, jnp.int32)])
        w = plsc.bitcast(wbits, jnp.float32)
        for g in range(DP // 16):
            x = pin[e, pl.ds(g * 16, 16)]
            flo = plsc.bitcast(lax.shift_left(x, 16), jnp.float32) * w
            fhi = plsc.bitcast(jnp.bitwise_and(x, jnp.int32(-65536)),
                               jnp.float32) * w
            rout[e, pl.ds(g * 16, 16)] = flo
            rout[e, pl.ds(DP + g * 16, 16)] = fhi


def _scale_rows(rows, wrow_ref):
    @pl.loop(0, CHUNK, unroll=4)
    def _edge(e):
        wbits = plsc.load_gather(wrow_ref, [jnp.full((16,), e, jnp.int32)])
        wsplat = plsc.bitcast(wbits, jnp.float32)
        for c in range(D // 16):
            sl = (e, pl.ds(c * 16, 16))
            rows[sl] = rows[sl] * wsplat


def _sc_body(hlin_hbm, hlinp_hbm, meta_hbm, zeros_hbm, out_hbm,
             r0, r1, r2, p0, p1, p2, m0, m1, m2, m3, m4, m5, acc,
             g0, g1, g2, s0, s1, s2, n0, n1, n2, n3, n4, n5,
             ):
    cid = lax.axis_index("c")
    sid = lax.axis_index("s")
    rows = (r0, r1, r2)
    pin = (p0, p1, p2)
    meta = (m0, m1, m2, m3, m4, m5)
    gsem = (g0, g1, g2)
    ssem = (s0, s1, s2)
    msem = (n0, n1, n2, n3, n4, n5)
    cpt = jnp.where(cid == 0, CPT0, CPT1)
    base = jnp.where(cid == 0, sid * CPT0, SC_TILES * CPT0 + sid * CPT1)

    row0 = sid * ROWS_PER_TILE
    pltpu.sync_copy(zeros_hbm, acc.at[pl.ds(row0, ROWS_PER_TILE)])
    plsc.subcore_barrier()

    def start_meta(ms, c):
        pltpu.async_copy(meta_hbm.at[base + c], meta[ms], msem[ms])

    def wait_meta(ms, c):
        pltpu.make_async_copy(meta_hbm.at[base + c], meta[ms], msem[ms]).wait()

    on_sc0 = cid == 0

    def start_gather(b, ms, c):
        @pl.when(on_sc0)
        def _():
            pltpu.async_copy(hlin_hbm.at[meta[ms].at[0]], rows[b], gsem[b])

        @pl.when(jnp.logical_not(on_sc0))
        def _():
            pltpu.async_copy(hlinp_hbm.at[meta[ms].at[0]], pin[b], gsem[b])

    def wait_gather(b, ms):
        @pl.when(on_sc0)
        def _():
            pltpu.make_async_copy(hlin_hbm.at[meta[ms].at[0]], rows[b],
                                  gsem[b]).wait()

        @pl.when(jnp.logical_not(on_sc0))
        def _():
            pltpu.make_async_copy(hlinp_hbm.at[meta[ms].at[0]], pin[b],
                                  gsem[b]).wait()

    def start_scatter(b, ms, c):
        pltpu.async_copy(rows[b], acc.at[meta[ms].at[1]], ssem[b], add=True)

    def wait_scatter(b, ms):
        pltpu.make_async_copy(rows[b], acc.at[meta[ms].at[1]], ssem[b]).wait()

    for ms in range(MSLOTS):
        pltpu.sync_copy(meta_hbm.at[base + ms], meta[ms])
    for b in range(LEAD):
        start_gather(b, b, b)

    @pl.loop(0, cpt, step=MSLOTS)
    def _group(k):
        for i in range(MSLOTS):
            c = k + i
            b = i % NBUF
            ms = i
            wait_gather(b, ms)

            @pl.when(on_sc0)
            def _():
                _scale_rows(rows[b], meta[ms].at[2])

            @pl.when(jnp.logical_not(on_sc0))
            def _():
                _expand_scale(pin[b], rows[b], meta[ms].at[2])

            start_scatter(b, ms, c)

            bn = (b + NBUF - 1) % NBUF
            msn = (i + LEAD) % MSLOTS
            msp = (i + MSLOTS - 1) % MSLOTS

            @pl.when(c == 0)
            def _():
                start_gather(bn, msn, LEAD)

            @pl.when(jnp.logical_and(c >= 1, c + LEAD < cpt))
            def _():
                wait_scatter(bn, msp)

                @pl.when(c + MSLOTS - 1 < cpt)
                def _():
                    start_meta(msp, c + MSLOTS - 1)

                @pl.when(c >= MSLOTS - LEAD)
                def _():
                    wait_meta(msn, c + LEAD)

                start_gather(bn, msn, c + LEAD)

    for b in range(NBUF):
        wait_scatter(b, b)

    plsc.subcore_barrier()
    pltpu.sync_copy(acc.at[pl.ds(row0, ROWS_PER_TILE)],
                    out_hbm.at[cid, pl.ds(row0, ROWS_PER_TILE)])


_SC_PARAMS = pltpu.CompilerParams()
for _f, _v in (("needs_layout_passes", False), ("use_tc_tiling_on_sc", False)):
    if _f in pltpu.CompilerParams.__dataclass_fields__:
        _SC_PARAMS = dataclasses.replace(_SC_PARAMS, **{_f: _v})


def _sc_segment(hlin, hlin_packed, meta, zeros_rows):
    kern = pl.kernel(
        _sc_body,
        out_type=jax.ShapeDtypeStruct((2, N_PAD, D), jnp.float32),
        mesh=plsc.VectorSubcoreMesh(core_axis_name="c", subcore_axis_name="s"),
        scratch_types=(
            [pltpu.VMEM((CHUNK, D), jnp.float32) for _ in range(NBUF)]
            + [pltpu.VMEM((CHUNK, DP), jnp.int32) for _ in range(NBUF)]
            + [pltpu.VMEM((3, CHUNK), jnp.int32) for _ in range(MSLOTS)]
            + [pltpu.VMEM_SHARED((N_PAD, D), jnp.float32)]
            + [pltpu.SemaphoreType.DMA for _ in range(NBUF + NBUF + MSLOTS)]
        ),
        compiler_params=_SC_PARAMS,
    )
    return kern(hlin, hlin_packed, meta, zeros_rows)


def _mm(a, w):
    return lax.dot_general(a, w, (((1,), (1,)), ((), ())),
                           preferred_element_type=jnp.float32)


def _pack_rows(h):
    u = lax.bitcast_convert_type(h, jnp.int32) + jnp.int32(0x8000)
    lo = lax.shift_right_logical(u[:, :DP], 16)
    hi = jnp.bitwise_and(u[:, DP:], jnp.int32(-65536))
    return jnp.bitwise_or(lo, hi)


def _in_body(x_ref, wi_ref, bi_ref, w0_ref, b0_ref, t_ref,
             hlin_ref, hlinp_ref, hid_ref):
    h = _mm(x_ref[...], wi_ref[...]) + bi_ref[...]
    hid_ref[...] = h * t_ref[0]
    hl = _mm(h, w0_ref[...]) + b0_ref[...]
    hlin_ref[...] = hl
    hlinp_ref[...] = _pack_rows(hl)


def _mid_body(p_ref, w_ref, b_ref, hin_ref, t_ref,
              hlin_ref, hlinp_ref, hout_ref, *, ti):
    h = jnp.maximum(p_ref[0] + p_ref[1], 0.0)
    hout_ref[...] = hin_ref[...] + h * t_ref[ti]
    hl = _mm(h, w_ref[...]) + b_ref[...]
    hlin_ref[...] = hl
    hlinp_ref[...] = _pack_rows(hl)


def _fin_body(p_ref, w_ref, b_ref, hin_ref, t_ref, out_ref, *, ti):
    h = jnp.maximum(p_ref[0] + p_ref[1], 0.0)
    hidden = hin_ref[...] + h * t_ref[ti]
    out_ref[...] = _mm(hidden, w_ref[...]) + b_ref[...]


_W_SPEC = pl.BlockSpec((D, D), lambda i: (0, 0))
_B_SPEC = pl.BlockSpec((1, D), lambda i: (0, 0))
_ROW_SPEC = pl.BlockSpec((ROW_BLOCK, D), lambda i: (i, 0))
_PKD_SPEC = pl.BlockSpec((ROW_BLOCK, DP), lambda i: (i, 0))
_P_SPEC = pl.BlockSpec((2, ROW_BLOCK, D), lambda i: (0, i, 0))
_T_SPEC = pl.BlockSpec(memory_space=pltpu.SMEM)


def _in_call(x, W_in, b_in, W0, b0, temp):
    n = x.shape[0]
    return pl.pallas_call(
        _in_body,
        grid=(n // ROW_BLOCK,),
        in_specs=[_ROW_SPEC, _W_SPEC, _B_SPEC, _W_SPEC, _B_SPEC, _T_SPEC],
        out_specs=[_ROW_SPEC, _PKD_SPEC, _ROW_SPEC],
        out_shape=[jax.ShapeDtypeStruct((n, D), jnp.float32),
                   jax.ShapeDtypeStruct((n, DP), jnp.int32),
                   jax.ShapeDtypeStruct((n, D), jnp.float32)],
    )(x, W_in, b_in, W0, b0, temp)


def _mid_call(parts, W, b, hidden, temp, ti):
    n = hidden.shape[0]
    return pl.pallas_call(
        functools.partial(_mid_body, ti=ti),
        grid=(n // ROW_BLOCK,),
        in_specs=[_P_SPEC, _W_SPEC, _B_SPEC, _ROW_SPEC, _T_SPEC],
        out_specs=[_ROW_SPEC, _PKD_SPEC, _ROW_SPEC],
        out_shape=[jax.ShapeDtypeStruct((n, D), jnp.float32),
                   jax.ShapeDtypeStruct((n, DP), jnp.int32),
                   jax.ShapeDtypeStruct((n, D), jnp.float32)],
    )(parts, W, b, hidden, temp)


def _fin_call(parts, W, b, hidden, temp, ti):
    n = hidden.shape[0]
    return pl.pallas_call(
        functools.partial(_fin_body, ti=ti),
        grid=(n // ROW_BLOCK,),
        in_specs=[_P_SPEC, _W_SPEC, _B_SPEC, _ROW_SPEC, _T_SPEC],
        out_specs=_ROW_SPEC,
        out_shape=jax.ShapeDtypeStruct((n, D), jnp.float32),
    )(parts, W, b, hidden, temp)


def kernel(x, edge_index, edge_w, W_in, b_in, W_layers, b_layers, W_out, b_out, temp):
    K = W_layers.shape[0]
    E = edge_index.shape[1]
    total_chunks = SC_TILES * (CPT0 + CPT1)
    e_pad = total_chunks * CHUNK
    pad = e_pad - E
    src = jnp.concatenate([edge_index[0], jnp.zeros((pad,), jnp.int32)])
    dst = jnp.concatenate([edge_index[1], jnp.zeros((pad,), jnp.int32)])
    wp = jnp.concatenate([edge_w, jnp.zeros((pad,), jnp.float32)])
    meta = jnp.stack([src, dst, lax.bitcast_convert_type(wp, jnp.int32)])
    meta = meta.reshape(3, total_chunks, CHUNK).transpose(1, 0, 2)
    zeros_rows = jnp.zeros((ROWS_PER_TILE, D), jnp.float32)

    hlin, hlinp, hidden = _in_call(x, W_in, b_in.reshape(1, D),
                                   W_layers[0], b_layers[0].reshape(1, D), temp)
    out = None
    for i in range(K):
        parts = _sc_segment(hlin, hlinp, meta, zeros_rows)
        if i < K - 1:
            hlin, hlinp, hidden = _mid_call(parts, W_layers[i + 1],
                                            b_layers[i + 1].reshape(1, D),
                                            hidden, temp, i + 1)
        else:
            out = _fin_call(parts, W_out, b_out.reshape(1, D),
                            hidden, temp, i + 1)
    return out
